# Initial kernel scaffold; baseline (speedup 1.0000x reference)
import jax
import jax.numpy as jnp
from jax.experimental import pallas as pl


def kernel(text_indices, char_emb, w1, b1, w2, b2, w3, b3, codebook, dw1, db1, dw2, db2, dw3, db3, W_out, b_out):
    raise NotImplementedError("write your pallas kernel here")


# speedup vs baseline: 4.8171x; 4.8171x over previous
import jax
import jax.numpy as jnp
from jax.experimental import pallas as pl

B = 512
VOCAB = 128
MAXLEN = 128
CED = 32
LAT = 64
H = 64
NCODES = 64
NSEQ = 30

BB = 8


def _relu(x):
    return jnp.maximum(x, 0.0)


def _body(t0r, t1r, t2r, t3r, cer,
          a0r, a1r, a2r, b1r,
          w0r, w1r, w2r, b2r,
          c0r, c1r, c2r, b3r,
          cbtr, cbr,
          d0r, d1r, d2r, db1r,
          e0r, e1r, e2r, e3r, db2r,
          f0r, f1r, f2r, f3r, db3r,
          wotr, borr,
          logits_ref, puz_ref, feat_ref, quant_ref, loss_ref):
    ce = cer[...]

    def embed(tr):
        t3d = tr[...]
        iota = jax.lax.broadcasted_iota(jnp.int32, (BB, 32, VOCAB), 2)
        oh = (t3d == iota).astype(jnp.float32).reshape(BB * 32, VOCAB)
        return jnp.dot(oh, ce, preferred_element_type=jnp.float32)

    x0, x1, x2, x3 = embed(t0r), embed(t1r), embed(t2r), embed(t3r)

    def shift_down(x3d):
        z = jnp.zeros((BB, 1, x3d.shape[2]), jnp.float32)
        return jnp.concatenate([z, x3d[:, :-1, :]], axis=1)

    b1 = b1r[...]
    x3s = shift_down(x3.reshape(BB, 32, CED)).reshape(BB * 32, CED)
    ye = _relu(x3s @ a0r[...] + x0 @ a1r[...] + x1 @ a2r[...] + b1)
    yo = _relu(x1 @ a0r[...] + x2 @ a1r[...] + x3 @ a2r[...] + b1)

    yos = shift_down(yo.reshape(BB, 32, H)).reshape(BB * 32, H)
    z = _relu(yos @ w0r[...] + ye @ w1r[...] + yo @ w2r[...] + b2r[...])
    z3 = z.reshape(BB, 32, H)

    enc = (z3[:, 0:30, :].reshape(BB * NSEQ, H) @ c0r[...]
           + z3[:, 1:31, :].reshape(BB * NSEQ, H) @ c1r[...]
           + z3[:, 2:32, :].reshape(BB * NSEQ, H) @ c2r[...]
           + b3r[...])

    cbt = cbtr[...]
    cbn = jnp.sum(cbt * cbt, axis=0, keepdims=True)
    en2 = jnp.sum(enc * enc, axis=1, keepdims=True)
    d2 = en2 + cbn - 2.0 * jnp.dot(enc, cbt, preferred_element_type=jnp.float32)
    dist = jnp.sqrt(jnp.maximum(d2, 0.0))
    mind = jnp.min(dist, axis=1, keepdims=True)
    idx = jax.lax.broadcasted_iota(jnp.int32, (BB * NSEQ, NCODES), 1)
    cand = jnp.where(dist == mind, idx, NCODES)
    codes = jnp.min(cand, axis=1, keepdims=True)
    ohq = (codes == idx).astype(jnp.float32)
    q = jnp.dot(ohq, cbr[...], preferred_element_type=jnp.float32)
    qst = enc + (q - enc)

    enc3 = enc.reshape(BB, NSEQ, LAT)
    qst3 = qst.reshape(BB, NSEQ, LAT)
    feat_ref[...] = jnp.broadcast_to(enc3[:, :, None, :], (BB, NSEQ, NSEQ, LAT))
    quant_ref[...] = jnp.broadcast_to(qst3[:, :, None, :], (BB, NSEQ, NSEQ, LAT))
    codes3 = codes.reshape(BB, NSEQ, 1)
    puz_ref[...] = jnp.broadcast_to(codes3, (BB, NSEQ, NSEQ))

    part = jnp.sum((enc - q) ** 2) * (1.25 / (B * NSEQ * LAT))

    @pl.when(pl.program_id(0) == 0)
    def _():
        loss_ref[...] = jnp.zeros_like(loss_ref)

    loss_ref[...] += part

    z2 = jnp.zeros((BB, 2, LAT), jnp.float32)
    buf = jnp.concatenate([z2, qst3, z2], axis=1)
    dd1 = _relu(buf[:, 2:34, :].reshape(BB * 32, LAT) @ d0r[...]
                + buf[:, 1:33, :].reshape(BB * 32, LAT) @ d1r[...]
                + buf[:, 0:32, :].reshape(BB * 32, LAT) @ d2r[...]
                + db1r[...])
    d13 = dd1.reshape(BB, 32, H)

    z1 = jnp.zeros((BB, 1, H), jnp.float32)
    buf2 = jnp.concatenate([z1, d13, z1], axis=1)
    db2 = db2r[...]
    ev = _relu(buf2[:, 1:33, :].reshape(BB * 32, H) @ e1r[...]
               + buf2[:, 0:32, :].reshape(BB * 32, H) @ e3r[...]
               + db2)
    od = _relu(buf2[:, 2:34, :].reshape(BB * 32, H) @ e0r[...]
               + buf2[:, 1:33, :].reshape(BB * 32, H) @ e2r[...]
               + db2)
    ev3 = ev.reshape(BB, 32, H)
    od3 = od.reshape(BB, 32, H)
    od_prev = jnp.concatenate([z1, od3[:, :31, :]], axis=1).reshape(BB * 32, H)
    ev_next = jnp.concatenate([ev3[:, 1:32, :], z1], axis=1).reshape(BB * 32, H)

    db3 = db3r[...]
    ph0 = ev @ f1r[...] + od_prev @ f3r[...] + db3
    ph1 = od @ f0r[...] + ev @ f2r[...] + db3
    ph2 = od @ f1r[...] + ev @ f3r[...] + db3
    ph3 = ev_next @ f0r[...] + od @ f2r[...] + db3

    wot = wotr[...]
    bo = borr[...]
    for k, ph in enumerate((ph0, ph1, ph2, ph3)):
        lg = jnp.dot(ph, wot, preferred_element_type=jnp.float32) + bo
        logits_ref[:, :, k, :] = lg.reshape(BB, 32, VOCAB)


def kernel(text_indices, char_emb, w1, b1, w2, b2, w3, b3, codebook,
           dw1, db1, dw2, db2, dw3, db3, W_out, b_out):
    t4 = text_indices.reshape(B, 32, 4)
    tph = [t4[:, :, r][:, :, None] for r in range(4)]

    A = [jnp.transpose(w1[:, :, k]) for k in range(3)]
    W2 = [jnp.transpose(w2[:, :, k]) for k in range(3)]
    C = [jnp.transpose(w3[:, :, k]) for k in range(3)]
    cbT = jnp.transpose(codebook)
    D = [dw1[:, :, k] for k in range(3)]
    E = [dw2[:, :, k] for k in range(4)]
    F = [dw3[:, :, k] for k in range(4)]
    WoT = jnp.transpose(W_out)

    row = lambda v: v[None, :]
    inputs = (
        tph[0], tph[1], tph[2], tph[3], char_emb,
        A[0], A[1], A[2], row(b1),
        W2[0], W2[1], W2[2], row(b2),
        C[0], C[1], C[2], row(b3),
        cbT, codebook,
        D[0], D[1], D[2], row(db1),
        E[0], E[1], E[2], E[3], row(db2),
        F[0], F[1], F[2], F[3], row(db3),
        WoT, row(b_out),
    )

    G = B // BB
    full = lambda shape: pl.BlockSpec(shape, lambda i: (0,) * len(shape))
    in_specs = [
        pl.BlockSpec((BB, 32, 1), lambda i: (i, 0, 0)),
        pl.BlockSpec((BB, 32, 1), lambda i: (i, 0, 0)),
        pl.BlockSpec((BB, 32, 1), lambda i: (i, 0, 0)),
        pl.BlockSpec((BB, 32, 1), lambda i: (i, 0, 0)),
    ] + [full(x.shape) for x in inputs[4:]]

    out_shapes = (
        jax.ShapeDtypeStruct((B, 32, 4, VOCAB), jnp.float32),
        jax.ShapeDtypeStruct((B, NSEQ, NSEQ), jnp.int32),
        jax.ShapeDtypeStruct((B, NSEQ, NSEQ, LAT), jnp.float32),
        jax.ShapeDtypeStruct((B, NSEQ, NSEQ, LAT), jnp.float32),
        jax.ShapeDtypeStruct((1, 1), jnp.float32),
    )
    out_specs = (
        pl.BlockSpec((BB, 32, 4, VOCAB), lambda i: (i, 0, 0, 0)),
        pl.BlockSpec((BB, NSEQ, NSEQ), lambda i: (i, 0, 0)),
        pl.BlockSpec((BB, NSEQ, NSEQ, LAT), lambda i: (i, 0, 0, 0)),
        pl.BlockSpec((BB, NSEQ, NSEQ, LAT), lambda i: (i, 0, 0, 0)),
        pl.BlockSpec((1, 1), lambda i: (0, 0)),
    )

    logits4, puzzles, features, quantized, loss = pl.pallas_call(
        _body,
        grid=(G,),
        in_specs=in_specs,
        out_specs=out_specs,
        out_shape=out_shapes,
    )(*inputs)

    char_logits = logits4.reshape(B, MAXLEN, VOCAB)
    vq_loss = loss.reshape(())
    return char_logits, puzzles, features, quantized, vq_loss


# speedup vs baseline: 7.0709x; 1.4679x over previous
import jax
import jax.numpy as jnp
from jax.experimental import pallas as pl

B = 512
VOCAB = 128
MAXLEN = 128
CED = 32
LAT = 64
H = 64
NCODES = 64
NSEQ = 30

BB = 16


def _relu(x):
    return jnp.maximum(x, 0.0)


def _body(t0r, t1r, t2r, t3r, cer,
          a0r, a1r, a2r, b1r,
          w0r, w1r, w2r, b2r,
          c0r, c1r, c2r, b3r,
          cbtr, cbr,
          d0r, d1r, d2r, db1r,
          e0r, e1r, e2r, e3r, db2r,
          f0r, f1r, f2r, f3r, db3r,
          wotr, borr,
          logits_ref, puz_ref, feat_ref, quant_ref, loss_ref):
    ce = cer[...]

    def embed(tr):
        t3d = tr[...]
        iota = jax.lax.broadcasted_iota(jnp.int32, (BB, 32, VOCAB), 2)
        oh = (t3d == iota).astype(jnp.float32).reshape(BB * 32, VOCAB)
        return jnp.dot(oh, ce, preferred_element_type=jnp.float32)

    x0, x1, x2, x3 = embed(t0r), embed(t1r), embed(t2r), embed(t3r)

    def shift_down(x3d):
        z = jnp.zeros((BB, 1, x3d.shape[2]), jnp.float32)
        return jnp.concatenate([z, x3d[:, :-1, :]], axis=1)

    b1 = b1r[...]
    x3s = shift_down(x3.reshape(BB, 32, CED)).reshape(BB * 32, CED)
    ye = _relu(x3s @ a0r[...] + x0 @ a1r[...] + x1 @ a2r[...] + b1)
    yo = _relu(x1 @ a0r[...] + x2 @ a1r[...] + x3 @ a2r[...] + b1)

    yos = shift_down(yo.reshape(BB, 32, H)).reshape(BB * 32, H)
    z = _relu(yos @ w0r[...] + ye @ w1r[...] + yo @ w2r[...] + b2r[...])
    z3 = z.reshape(BB, 32, H)

    enc = (z3[:, 0:30, :].reshape(BB * NSEQ, H) @ c0r[...]
           + z3[:, 1:31, :].reshape(BB * NSEQ, H) @ c1r[...]
           + z3[:, 2:32, :].reshape(BB * NSEQ, H) @ c2r[...]
           + b3r[...])

    cbt = cbtr[...]
    cbn = jnp.sum(cbt * cbt, axis=0, keepdims=True)
    en2 = jnp.sum(enc * enc, axis=1, keepdims=True)
    d2 = en2 + cbn - 2.0 * jnp.dot(enc, cbt, preferred_element_type=jnp.float32)
    dist = jnp.sqrt(jnp.maximum(d2, 0.0))
    mind = jnp.min(dist, axis=1, keepdims=True)
    idx = jax.lax.broadcasted_iota(jnp.int32, (BB * NSEQ, NCODES), 1)
    cand = jnp.where(dist == mind, idx, NCODES)
    codes = jnp.min(cand, axis=1, keepdims=True)
    ohq = (codes == idx).astype(jnp.float32)
    q = jnp.dot(ohq, cbr[...], preferred_element_type=jnp.float32)
    qst = enc + (q - enc)

    enc3 = enc.reshape(BB, NSEQ, LAT)
    qst3 = qst.reshape(BB, NSEQ, LAT)
    encp = jnp.concatenate([enc3, enc3], axis=2)
    qstp = jnp.concatenate([qst3, qst3], axis=2)
    feat_ref[...] = jnp.broadcast_to(encp[:, :, None, :], (BB, NSEQ, NSEQ // 2, 2 * LAT))
    quant_ref[...] = jnp.broadcast_to(qstp[:, :, None, :], (BB, NSEQ, NSEQ // 2, 2 * LAT))
    codes3 = codes.reshape(BB, NSEQ, 1)
    puz_ref[...] = jnp.broadcast_to(codes3, (BB, NSEQ, NSEQ))

    part = jnp.sum((enc - q) ** 2) * (1.25 / (B * NSEQ * LAT))

    @pl.when(pl.program_id(0) == 0)
    def _():
        loss_ref[...] = jnp.zeros_like(loss_ref)

    loss_ref[...] += part

    z2 = jnp.zeros((BB, 2, LAT), jnp.float32)
    buf = jnp.concatenate([z2, qst3, z2], axis=1)
    dd1 = _relu(buf[:, 2:34, :].reshape(BB * 32, LAT) @ d0r[...]
                + buf[:, 1:33, :].reshape(BB * 32, LAT) @ d1r[...]
                + buf[:, 0:32, :].reshape(BB * 32, LAT) @ d2r[...]
                + db1r[...])
    d13 = dd1.reshape(BB, 32, H)

    z1 = jnp.zeros((BB, 1, H), jnp.float32)
    buf2 = jnp.concatenate([z1, d13, z1], axis=1)
    db2 = db2r[...]
    ev = _relu(buf2[:, 1:33, :].reshape(BB * 32, H) @ e1r[...]
               + buf2[:, 0:32, :].reshape(BB * 32, H) @ e3r[...]
               + db2)
    od = _relu(buf2[:, 2:34, :].reshape(BB * 32, H) @ e0r[...]
               + buf2[:, 1:33, :].reshape(BB * 32, H) @ e2r[...]
               + db2)
    ev3 = ev.reshape(BB, 32, H)
    od3 = od.reshape(BB, 32, H)
    od_prev = jnp.concatenate([z1, od3[:, :31, :]], axis=1).reshape(BB * 32, H)
    ev_next = jnp.concatenate([ev3[:, 1:32, :], z1], axis=1).reshape(BB * 32, H)

    db3 = db3r[...]
    ph0 = ev @ f1r[...] + od_prev @ f3r[...] + db3
    ph1 = od @ f0r[...] + ev @ f2r[...] + db3
    ph2 = od @ f1r[...] + ev @ f3r[...] + db3
    ph3 = ev_next @ f0r[...] + od @ f2r[...] + db3

    wot = wotr[...]
    bo = borr[...]
    lgs = [jnp.dot(ph, wot, preferred_element_type=jnp.float32) + bo
           for ph in (ph0, ph1, ph2, ph3)]
    logits_ref[...] = jnp.concatenate(lgs, axis=1).reshape(BB, 32, 4 * VOCAB)


def kernel(text_indices, char_emb, w1, b1, w2, b2, w3, b3, codebook,
           dw1, db1, dw2, db2, dw3, db3, W_out, b_out):
    t4 = text_indices.reshape(B, 32, 4)
    tph = [t4[:, :, r][:, :, None] for r in range(4)]

    A = [jnp.transpose(w1[:, :, k]) for k in range(3)]
    W2 = [jnp.transpose(w2[:, :, k]) for k in range(3)]
    C = [jnp.transpose(w3[:, :, k]) for k in range(3)]
    cbT = jnp.transpose(codebook)
    D = [dw1[:, :, k] for k in range(3)]
    E = [dw2[:, :, k] for k in range(4)]
    F = [dw3[:, :, k] for k in range(4)]
    WoT = jnp.transpose(W_out)

    row = lambda v: v[None, :]
    inputs = (
        tph[0], tph[1], tph[2], tph[3], char_emb,
        A[0], A[1], A[2], row(b1),
        W2[0], W2[1], W2[2], row(b2),
        C[0], C[1], C[2], row(b3),
        cbT, codebook,
        D[0], D[1], D[2], row(db1),
        E[0], E[1], E[2], E[3], row(db2),
        F[0], F[1], F[2], F[3], row(db3),
        WoT, row(b_out),
    )

    G = B // BB
    full = lambda shape: pl.BlockSpec(shape, lambda i: (0,) * len(shape))
    in_specs = [
        pl.BlockSpec((BB, 32, 1), lambda i: (i, 0, 0)),
        pl.BlockSpec((BB, 32, 1), lambda i: (i, 0, 0)),
        pl.BlockSpec((BB, 32, 1), lambda i: (i, 0, 0)),
        pl.BlockSpec((BB, 32, 1), lambda i: (i, 0, 0)),
    ] + [full(x.shape) for x in inputs[4:]]

    out_shapes = (
        jax.ShapeDtypeStruct((B, 32, 4 * VOCAB), jnp.float32),
        jax.ShapeDtypeStruct((B, NSEQ, NSEQ), jnp.int32),
        jax.ShapeDtypeStruct((B, NSEQ, NSEQ // 2, 2 * LAT), jnp.float32),
        jax.ShapeDtypeStruct((B, NSEQ, NSEQ // 2, 2 * LAT), jnp.float32),
        jax.ShapeDtypeStruct((1, 1), jnp.float32),
    )
    out_specs = (
        pl.BlockSpec((BB, 32, 4 * VOCAB), lambda i: (i, 0, 0)),
        pl.BlockSpec((BB, NSEQ, NSEQ), lambda i: (i, 0, 0)),
        pl.BlockSpec((BB, NSEQ, NSEQ // 2, 2 * LAT), lambda i: (i, 0, 0, 0)),
        pl.BlockSpec((BB, NSEQ, NSEQ // 2, 2 * LAT), lambda i: (i, 0, 0, 0)),
        pl.BlockSpec((1, 1), lambda i: (0, 0)),
    )

    logits4, puzzles, features, quantized, loss = pl.pallas_call(
        _body,
        grid=(G,),
        in_specs=in_specs,
        out_specs=out_specs,
        out_shape=out_shapes,
    )(*inputs)

    char_logits = logits4.reshape(B, MAXLEN, VOCAB)
    features = features.reshape(B, NSEQ, NSEQ, LAT)
    quantized = quantized.reshape(B, NSEQ, NSEQ, LAT)
    vq_loss = loss.reshape(())
    return char_logits, puzzles, features, quantized, vq_loss
